# Initial kernel scaffold; baseline (speedup 1.0000x reference)
import jax
import jax.numpy as jnp
from jax.experimental import pallas as pl


def kernel(x, rbf, i, num_nodes, W_rbf, W1, b1, W2, b2, W3, b3, W_out):
    raise NotImplementedError("write your pallas kernel here")


# speedup vs baseline: 2.6164x; 2.6164x over previous
import functools

import jax
import jax.numpy as jnp
from jax import lax
from jax.experimental import pallas as pl
from jax.experimental.pallas import tpu as pltpu
from jax.experimental.pallas import tpu_sc as plsc

N_NODES = 10000
N_PAD = 10240
HIDDEN = 128

_BE = 3200


def _edge_body(rbf_ref, x_ref, wrbf_ref, h_ref):
    h_ref[...] = (
        jnp.dot(rbf_ref[...], wrbf_ref[...], preferred_element_type=jnp.float32)
        * x_ref[...]
    )


def _edge_stage(rbf, x, w_rbf):
    e = x.shape[0]
    return pl.pallas_call(
        _edge_body,
        grid=(e // _BE,),
        in_specs=[
            pl.BlockSpec((_BE, rbf.shape[1]), lambda g: (g, 0)),
            pl.BlockSpec((_BE, HIDDEN), lambda g: (g, 0)),
            pl.BlockSpec((rbf.shape[1], HIDDEN), lambda g: (0, 0)),
        ],
        out_specs=pl.BlockSpec((_BE, HIDDEN), lambda g: (g, 0)),
        out_shape=jax.ShapeDtypeStruct((e, HIDDEN), jnp.float32),
    )(rbf, x, w_rbf)


_CHUNK = 256
_IDXROW = 128


def _sc_scatter(h, seg2d, zeros_nf):
    e = h.shape[0]
    nchunks = e // _CHUNK
    nworkers = 32
    trips = (nchunks + nworkers - 1) // nworkers
    mesh = plsc.VectorSubcoreMesh(core_axis_name="c", subcore_axis_name="s")
    rows_per_tile = N_PAD // 16

    @functools.partial(
        pl.kernel,
        out_type=jax.ShapeDtypeStruct((2, N_PAD, HIDDEN), jnp.float32),
        mesh=mesh,
        scratch_types=[
            pltpu.VMEM_SHARED((N_PAD, HIDDEN), jnp.float32),
            pltpu.VMEM((_CHUNK, HIDDEN), jnp.float32),
            pltpu.VMEM((_CHUNK // _IDXROW, _IDXROW), jnp.int32),
        ],
    )
    def scatter_kernel(h_hbm, idx_hbm, zeros_hbm, out_hbm, acc_sh, rows_v, idx_v):
        c = lax.axis_index("c")
        s = lax.axis_index("s")
        w = c * 16 + s
        pltpu.sync_copy(
            zeros_hbm.at[pl.ds(s * rows_per_tile, rows_per_tile)],
            acc_sh.at[pl.ds(s * rows_per_tile, rows_per_tile)],
        )
        plsc.subcore_barrier()

        def trip(t, carry):
            g = w + t * nworkers

            @pl.when(g < nchunks)
            def _():
                pltpu.sync_copy(h_hbm.at[pl.ds(g * _CHUNK, _CHUNK)], rows_v)
                pltpu.sync_copy(
                    idx_hbm.at[pl.ds(g * (_CHUNK // _IDXROW), _CHUNK // _IDXROW)],
                    idx_v,
                )
                for j in range(_CHUNK // _IDXROW):
                    pltpu.sync_copy(
                        rows_v.at[pl.ds(j * _IDXROW, _IDXROW)],
                        acc_sh.at[idx_v.at[j]],
                        add=True,
                    )

            return carry

        lax.fori_loop(0, trips, trip, 0)
        plsc.subcore_barrier()
        pltpu.sync_copy(
            acc_sh.at[pl.ds(s * rows_per_tile, rows_per_tile)],
            out_hbm.at[c].at[pl.ds(s * rows_per_tile, rows_per_tile)],
        )

    return scatter_kernel(h, seg2d, zeros_nf)


_BN = 1000


def _mlp_body(p_ref, w1_ref, b1_ref, w2_ref, b2_ref, w3_ref, b3_ref, wo_ref, o_ref):
    a = p_ref[0] + p_ref[1]

    def sw(v):
        return v * (1.0 / (1.0 + jnp.exp(-v)))

    a = sw(jnp.dot(a, w1_ref[...], preferred_element_type=jnp.float32) + b1_ref[...])
    a = sw(jnp.dot(a, w2_ref[...], preferred_element_type=jnp.float32) + b2_ref[...])
    a = sw(jnp.dot(a, w3_ref[...], preferred_element_type=jnp.float32) + b3_ref[...])
    o_ref[...] = jnp.dot(a, wo_ref[...], preferred_element_type=jnp.float32)


def _mlp_stage(parts, w1, b1, w2, b2, w3, b3, w_out):
    full = lambda r, c: pl.BlockSpec((r, c), lambda g: (0, 0))
    return pl.pallas_call(
        _mlp_body,
        grid=(N_NODES // _BN,),
        in_specs=[
            pl.BlockSpec((2, _BN, HIDDEN), lambda g: (0, g, 0)),
            full(HIDDEN, HIDDEN), full(1, HIDDEN),
            full(HIDDEN, HIDDEN), full(1, HIDDEN),
            full(HIDDEN, HIDDEN), full(1, HIDDEN),
            full(HIDDEN, w_out.shape[1]),
        ],
        out_specs=pl.BlockSpec((_BN, w_out.shape[1]), lambda g: (g, 0)),
        out_shape=jax.ShapeDtypeStruct((N_NODES, w_out.shape[1]), jnp.float32),
    )(parts, w1, b1.reshape(1, -1), w2, b2.reshape(1, -1),
      w3, b3.reshape(1, -1), w_out)


def kernel(x, rbf, i, num_nodes, W_rbf, W1, b1, W2, b2, W3, b3, W_out):
    seg = jnp.minimum(i.astype(jnp.int32), jnp.int32(num_nodes) - 1)
    seg2d = seg.reshape(-1, _IDXROW)
    h = _edge_stage(rbf, x, W_rbf)
    zeros_nf = jnp.zeros((N_PAD, HIDDEN), jnp.float32)
    parts = _sc_scatter(h, seg2d, zeros_nf)
    return _mlp_stage(parts, W1, b1, W2, b2, W3, b3, W_out)


# speedup vs baseline: 3.0493x; 1.1654x over previous
import functools

import jax
import jax.numpy as jnp
from jax import lax
from jax.experimental import pallas as pl
from jax.experimental.pallas import tpu as pltpu
from jax.experimental.pallas import tpu_sc as plsc

N_NODES = 10000
N_PAD = 10240
HIDDEN = 128

_BE = 3200


def _edge_body(rbf_ref, x_ref, wrbf_ref, h_ref):
    h_ref[...] = (
        jnp.dot(rbf_ref[...], wrbf_ref[...], preferred_element_type=jnp.float32)
        * x_ref[...]
    )


def _edge_stage(rbf, x, w_rbf):
    e = x.shape[0]
    return pl.pallas_call(
        _edge_body,
        grid=(e // _BE,),
        in_specs=[
            pl.BlockSpec((_BE, rbf.shape[1]), lambda g: (g, 0)),
            pl.BlockSpec((_BE, HIDDEN), lambda g: (g, 0)),
            pl.BlockSpec((rbf.shape[1], HIDDEN), lambda g: (0, 0)),
        ],
        out_specs=pl.BlockSpec((_BE, HIDDEN), lambda g: (g, 0)),
        out_shape=jax.ShapeDtypeStruct((e, HIDDEN), jnp.float32),
    )(rbf, x, w_rbf)


_CHUNK = 128
_IDXROW = 128


def _sc_scatter(h, seg2d, zeros_nf):
    e = h.shape[0]
    nchunks = e // _CHUNK
    nworkers = 32
    trips = (nchunks + 2 * nworkers - 1) // (2 * nworkers)
    mesh = plsc.VectorSubcoreMesh(core_axis_name="c", subcore_axis_name="s")
    rows_per_tile = N_PAD // 16

    @functools.partial(
        pl.kernel,
        out_type=jax.ShapeDtypeStruct((2, N_PAD, HIDDEN), jnp.float32),
        mesh=mesh,
        scratch_types=[
            pltpu.VMEM_SHARED((N_PAD, HIDDEN), jnp.float32),
            pltpu.VMEM((2, _CHUNK, HIDDEN), jnp.float32),
            pltpu.VMEM((2, _IDXROW), jnp.int32),
            pltpu.SemaphoreType.DMA,
            pltpu.SemaphoreType.DMA,
        ],
    )
    def scatter_kernel(h_hbm, idx_hbm, zeros_hbm, out_hbm, acc_sh, rows_v,
                       idx_v, sem0, sem1):
        c = lax.axis_index("c")
        s = lax.axis_index("s")
        w = c * 16 + s
        sems = (sem0, sem1)
        pltpu.sync_copy(
            zeros_hbm.at[pl.ds(s * rows_per_tile, rows_per_tile)],
            acc_sh.at[pl.ds(s * rows_per_tile, rows_per_tile)],
        )
        plsc.subcore_barrier()

        def chunk_of(k):
            return w + k * nworkers

        def load(k, b):
            g = chunk_of(k)

            @pl.when(g < nchunks)
            def _():
                pltpu.async_copy(
                    h_hbm.at[pl.ds(g * _CHUNK, _CHUNK)], rows_v.at[b], sems[b]
                )
                pltpu.async_copy(
                    idx_hbm.at[pl.ds(g, 1)], idx_v.at[pl.ds(b, 1)], sems[b]
                )

        def drain_scatter(k, b):
            g = chunk_of(k)

            @pl.when(g < nchunks)
            def _():
                pltpu.make_async_copy(
                    h_hbm.at[pl.ds(g * _CHUNK, _CHUNK)], rows_v.at[b], sems[b]
                ).wait()
                pltpu.make_async_copy(
                    idx_hbm.at[pl.ds(g, 1)], idx_v.at[pl.ds(b, 1)], sems[b]
                ).wait()
                pltpu.sync_copy(rows_v.at[b], acc_sh.at[idx_v.at[b]], add=True)

        load(0, 0)

        def trip(t, carry):
            load(2 * t + 1, 1)
            drain_scatter(2 * t, 0)
            load(2 * t + 2, 0)
            drain_scatter(2 * t + 1, 1)
            return carry

        lax.fori_loop(0, trips, trip, 0)
        plsc.subcore_barrier()
        pltpu.sync_copy(
            acc_sh.at[pl.ds(s * rows_per_tile, rows_per_tile)],
            out_hbm.at[c].at[pl.ds(s * rows_per_tile, rows_per_tile)],
        )

    return scatter_kernel(h, seg2d, zeros_nf)


_BN = 1000


def _mlp_body(p_ref, w1_ref, b1_ref, w2_ref, b2_ref, w3_ref, b3_ref, wo_ref, o_ref):
    a = p_ref[0] + p_ref[1]

    def sw(v):
        return v * (1.0 / (1.0 + jnp.exp(-v)))

    a = sw(jnp.dot(a, w1_ref[...], preferred_element_type=jnp.float32) + b1_ref[...])
    a = sw(jnp.dot(a, w2_ref[...], preferred_element_type=jnp.float32) + b2_ref[...])
    a = sw(jnp.dot(a, w3_ref[...], preferred_element_type=jnp.float32) + b3_ref[...])
    o_ref[...] = jnp.dot(a, wo_ref[...], preferred_element_type=jnp.float32)


def _mlp_stage(parts, w1, b1, w2, b2, w3, b3, w_out):
    full = lambda r, c: pl.BlockSpec((r, c), lambda g: (0, 0))
    return pl.pallas_call(
        _mlp_body,
        grid=(N_NODES // _BN,),
        in_specs=[
            pl.BlockSpec((2, _BN, HIDDEN), lambda g: (0, g, 0)),
            full(HIDDEN, HIDDEN), full(1, HIDDEN),
            full(HIDDEN, HIDDEN), full(1, HIDDEN),
            full(HIDDEN, HIDDEN), full(1, HIDDEN),
            full(HIDDEN, w_out.shape[1]),
        ],
        out_specs=pl.BlockSpec((_BN, w_out.shape[1]), lambda g: (g, 0)),
        out_shape=jax.ShapeDtypeStruct((N_NODES, w_out.shape[1]), jnp.float32),
    )(parts, w1, b1.reshape(1, -1), w2, b2.reshape(1, -1),
      w3, b3.reshape(1, -1), w_out)


def kernel(x, rbf, i, num_nodes, W_rbf, W1, b1, W2, b2, W3, b3, W_out):
    seg = jnp.minimum(i.astype(jnp.int32), jnp.int32(num_nodes) - 1)
    seg2d = seg.reshape(-1, _IDXROW)
    h = _edge_stage(rbf, x, W_rbf)
    zeros_nf = jnp.zeros((N_PAD, HIDDEN), jnp.float32)
    parts = _sc_scatter(h, seg2d, zeros_nf)
    return _mlp_stage(parts, W1, b1, W2, b2, W3, b3, W_out)


# speedup vs baseline: 4.0113x; 1.3155x over previous
import functools

import jax
import jax.numpy as jnp
from jax import lax
from jax.experimental import pallas as pl
from jax.experimental.pallas import tpu as pltpu
from jax.experimental.pallas import tpu_sc as plsc

N_NODES = 10000
N_PAD = 10240
HIDDEN = 128

_BE = 3200


def _edge_body(rbft_ref, x_ref, wrbf_ref, h_ref):
    coeff = jax.lax.dot_general(
        rbft_ref[...],
        wrbf_ref[...],
        dimension_numbers=(((0,), (0,)), ((), ())),
        preferred_element_type=jnp.float32,
    )
    h_ref[...] = coeff * x_ref[...]


def _edge_stage(rbf_t, x, w_rbf):
    e = x.shape[0]
    nr = rbf_t.shape[0]
    return pl.pallas_call(
        _edge_body,
        grid=(e // _BE,),
        in_specs=[
            pl.BlockSpec((nr, _BE), lambda g: (0, g)),
            pl.BlockSpec((_BE, HIDDEN), lambda g: (g, 0)),
            pl.BlockSpec((nr, HIDDEN), lambda g: (0, 0)),
        ],
        out_specs=pl.BlockSpec((_BE, HIDDEN), lambda g: (g, 0)),
        out_shape=jax.ShapeDtypeStruct((e, HIDDEN), jnp.float32),
    )(rbf_t, x, w_rbf)


_CHUNK = 128
_IDXROW = 128


def _sc_scatter(h, seg2d, zeros_nf):
    e = h.shape[0]
    nchunks = e // _CHUNK
    nworkers = 32
    trips = (nchunks + 2 * nworkers - 1) // (2 * nworkers)
    mesh = plsc.VectorSubcoreMesh(core_axis_name="c", subcore_axis_name="s")
    rows_per_tile = N_PAD // 16

    @functools.partial(
        pl.kernel,
        out_type=jax.ShapeDtypeStruct((2, N_PAD, HIDDEN), jnp.float32),
        mesh=mesh,
        scratch_types=[
            pltpu.VMEM_SHARED((N_PAD, HIDDEN), jnp.float32),
            pltpu.VMEM((2, _CHUNK, HIDDEN), jnp.float32),
            pltpu.VMEM((2, _IDXROW), jnp.int32),
            pltpu.SemaphoreType.DMA,
            pltpu.SemaphoreType.DMA,
        ],
    )
    def scatter_kernel(h_hbm, idx_hbm, zeros_hbm, out_hbm, acc_sh, rows_v,
                       idx_v, sem0, sem1):
        c = lax.axis_index("c")
        s = lax.axis_index("s")
        w = c * 16 + s
        sems = (sem0, sem1)
        pltpu.sync_copy(
            zeros_hbm.at[pl.ds(s * rows_per_tile, rows_per_tile)],
            acc_sh.at[pl.ds(s * rows_per_tile, rows_per_tile)],
        )
        plsc.subcore_barrier()

        def chunk_of(k):
            return w + k * nworkers

        def load(k, b):
            g = chunk_of(k)

            @pl.when(g < nchunks)
            def _():
                pltpu.async_copy(
                    h_hbm.at[pl.ds(g * _CHUNK, _CHUNK)], rows_v.at[b], sems[b]
                )
                pltpu.async_copy(
                    idx_hbm.at[pl.ds(g, 1)], idx_v.at[pl.ds(b, 1)], sems[b]
                )

        def drain_scatter(k, b):
            g = chunk_of(k)

            @pl.when(g < nchunks)
            def _():
                pltpu.make_async_copy(
                    h_hbm.at[pl.ds(g * _CHUNK, _CHUNK)], rows_v.at[b], sems[b]
                ).wait()
                pltpu.make_async_copy(
                    idx_hbm.at[pl.ds(g, 1)], idx_v.at[pl.ds(b, 1)], sems[b]
                ).wait()
                pltpu.sync_copy(rows_v.at[b], acc_sh.at[idx_v.at[b]], add=True)

        load(0, 0)

        def trip(t, carry):
            load(2 * t + 1, 1)
            drain_scatter(2 * t, 0)
            load(2 * t + 2, 0)
            drain_scatter(2 * t + 1, 1)
            return carry

        lax.fori_loop(0, trips, trip, 0)
        plsc.subcore_barrier()
        pltpu.sync_copy(
            acc_sh.at[pl.ds(s * rows_per_tile, rows_per_tile)],
            out_hbm.at[c].at[pl.ds(s * rows_per_tile, rows_per_tile)],
        )

    return scatter_kernel(h, seg2d, zeros_nf)


_BN = 1000


def _mlp_body(p_ref, w1_ref, b1_ref, w2_ref, b2_ref, w3_ref, b3_ref, wo_ref, o_ref):
    a = p_ref[0] + p_ref[1]

    def sw(v):
        return v * (1.0 / (1.0 + jnp.exp(-v)))

    a = sw(jnp.dot(a, w1_ref[...], preferred_element_type=jnp.float32) + b1_ref[...])
    a = sw(jnp.dot(a, w2_ref[...], preferred_element_type=jnp.float32) + b2_ref[...])
    a = sw(jnp.dot(a, w3_ref[...], preferred_element_type=jnp.float32) + b3_ref[...])
    o_ref[...] = jnp.dot(a, wo_ref[...], preferred_element_type=jnp.float32)


def _mlp_stage(parts, w1, b1, w2, b2, w3, b3, w_out):
    full = lambda r, c: pl.BlockSpec((r, c), lambda g: (0, 0))
    return pl.pallas_call(
        _mlp_body,
        grid=(N_NODES // _BN,),
        in_specs=[
            pl.BlockSpec((2, _BN, HIDDEN), lambda g: (0, g, 0)),
            full(HIDDEN, HIDDEN), full(1, HIDDEN),
            full(HIDDEN, HIDDEN), full(1, HIDDEN),
            full(HIDDEN, HIDDEN), full(1, HIDDEN),
            full(HIDDEN, w_out.shape[1]),
        ],
        out_specs=pl.BlockSpec((_BN, w_out.shape[1]), lambda g: (g, 0)),
        out_shape=jax.ShapeDtypeStruct((N_NODES, w_out.shape[1]), jnp.float32),
    )(parts, w1, b1.reshape(1, -1), w2, b2.reshape(1, -1),
      w3, b3.reshape(1, -1), w_out)


def kernel(x, rbf, i, num_nodes, W_rbf, W1, b1, W2, b2, W3, b3, W_out):
    seg = jnp.minimum(i.astype(jnp.int32), jnp.int32(num_nodes) - 1)
    seg2d = seg.reshape(-1, _IDXROW)
    h = _edge_stage(rbf.T, x, W_rbf)
    zeros_nf = jnp.zeros((N_PAD, HIDDEN), jnp.float32)
    parts = _sc_scatter(h, seg2d, zeros_nf)
    return _mlp_stage(parts, W1, b1, W2, b2, W3, b3, W_out)


# speedup vs baseline: 4.3881x; 1.0939x over previous
import functools

import jax
import jax.numpy as jnp
from jax import lax
from jax.experimental import pallas as pl
from jax.experimental.pallas import tpu as pltpu
from jax.experimental.pallas import tpu_sc as plsc

N_NODES = 10000
N_PAD = 10240
HIDDEN = 128

_BE = 3200


def _edge_body(rbft_ref, x_ref, wrbf_ref, h_ref):
    coeff = jax.lax.dot_general(
        rbft_ref[...],
        wrbf_ref[...],
        dimension_numbers=(((0,), (0,)), ((), ())),
        preferred_element_type=jnp.float32,
    )
    h_ref[...] = coeff * x_ref[...]


def _edge_stage(rbf_t, x, w_rbf, off, e_slice):
    nr = rbf_t.shape[0]
    ob = off // _BE
    return pl.pallas_call(
        _edge_body,
        grid=(e_slice // _BE,),
        in_specs=[
            pl.BlockSpec((nr, _BE), lambda g: (0, g + ob)),
            pl.BlockSpec((_BE, HIDDEN), lambda g: (g + ob, 0)),
            pl.BlockSpec((nr, HIDDEN), lambda g: (0, 0)),
        ],
        out_specs=pl.BlockSpec((_BE, HIDDEN), lambda g: (g, 0)),
        out_shape=jax.ShapeDtypeStruct((e_slice, HIDDEN), jnp.float32),
    )(rbf_t, x, w_rbf)


_CHUNK = 128
_IDXROW = 128


def _sc_scatter(h, seg2d, zeros_nf):
    e = h.shape[0]
    nchunks = e // _CHUNK
    nworkers = 32
    trips = (nchunks + 2 * nworkers - 1) // (2 * nworkers)
    mesh = plsc.VectorSubcoreMesh(core_axis_name="c", subcore_axis_name="s")
    rows_per_tile = N_PAD // 16

    @functools.partial(
        pl.kernel,
        out_type=jax.ShapeDtypeStruct((2, N_PAD, HIDDEN), jnp.float32),
        mesh=mesh,
        scratch_types=[
            pltpu.VMEM_SHARED((N_PAD, HIDDEN), jnp.float32),
            pltpu.VMEM((2, _CHUNK, HIDDEN), jnp.float32),
            pltpu.VMEM((2, _IDXROW), jnp.int32),
            pltpu.SemaphoreType.DMA,
            pltpu.SemaphoreType.DMA,
        ],
    )
    def scatter_kernel(h_hbm, idx_hbm, zeros_hbm, out_hbm, acc_sh, rows_v,
                       idx_v, sem0, sem1):
        c = lax.axis_index("c")
        s = lax.axis_index("s")
        w = c * 16 + s
        sems = (sem0, sem1)
        pltpu.sync_copy(
            zeros_hbm.at[pl.ds(s * rows_per_tile, rows_per_tile)],
            acc_sh.at[pl.ds(s * rows_per_tile, rows_per_tile)],
        )
        plsc.subcore_barrier()

        def chunk_of(k):
            return w + k * nworkers

        def load(k, b):
            g = chunk_of(k)

            @pl.when(g < nchunks)
            def _():
                pltpu.async_copy(
                    h_hbm.at[pl.ds(g * _CHUNK, _CHUNK)], rows_v.at[b], sems[b]
                )
                pltpu.async_copy(
                    idx_hbm.at[pl.ds(g, 1)], idx_v.at[pl.ds(b, 1)], sems[b]
                )

        def drain_scatter(k, b):
            g = chunk_of(k)

            @pl.when(g < nchunks)
            def _():
                pltpu.make_async_copy(
                    h_hbm.at[pl.ds(g * _CHUNK, _CHUNK)], rows_v.at[b], sems[b]
                ).wait()
                pltpu.make_async_copy(
                    idx_hbm.at[pl.ds(g, 1)], idx_v.at[pl.ds(b, 1)], sems[b]
                ).wait()
                pltpu.sync_copy(rows_v.at[b], acc_sh.at[idx_v.at[b]], add=True)

        load(0, 0)

        def trip(t, carry):
            load(2 * t + 1, 1)
            drain_scatter(2 * t, 0)
            load(2 * t + 2, 0)
            drain_scatter(2 * t + 1, 1)
            return carry

        lax.fori_loop(0, trips, trip, 0)
        plsc.subcore_barrier()
        pltpu.sync_copy(
            acc_sh.at[pl.ds(s * rows_per_tile, rows_per_tile)],
            out_hbm.at[c].at[pl.ds(s * rows_per_tile, rows_per_tile)],
        )

    return scatter_kernel(h, seg2d, zeros_nf)


_BN = 1000


def _mlp_body(pa_ref, pb_ref, w1_ref, b1_ref, w2_ref, b2_ref, w3_ref, b3_ref,
              wo_ref, o_ref):
    a = (pa_ref[0] + pa_ref[1]) + (pb_ref[0] + pb_ref[1])

    def sw(v):
        return v * (1.0 / (1.0 + jnp.exp(-v)))

    a = sw(jnp.dot(a, w1_ref[...], preferred_element_type=jnp.float32) + b1_ref[...])
    a = sw(jnp.dot(a, w2_ref[...], preferred_element_type=jnp.float32) + b2_ref[...])
    a = sw(jnp.dot(a, w3_ref[...], preferred_element_type=jnp.float32) + b3_ref[...])
    o_ref[...] = jnp.dot(a, wo_ref[...], preferred_element_type=jnp.float32)


def _mlp_stage(parts_a, parts_b, w1, b1, w2, b2, w3, b3, w_out):
    full = lambda r, c: pl.BlockSpec((r, c), lambda g: (0, 0))
    return pl.pallas_call(
        _mlp_body,
        grid=(N_NODES // _BN,),
        in_specs=[
            pl.BlockSpec((2, _BN, HIDDEN), lambda g: (0, g, 0)),
            pl.BlockSpec((2, _BN, HIDDEN), lambda g: (0, g, 0)),
            full(HIDDEN, HIDDEN), full(1, HIDDEN),
            full(HIDDEN, HIDDEN), full(1, HIDDEN),
            full(HIDDEN, HIDDEN), full(1, HIDDEN),
            full(HIDDEN, w_out.shape[1]),
        ],
        out_specs=pl.BlockSpec((_BN, w_out.shape[1]), lambda g: (g, 0)),
        out_shape=jax.ShapeDtypeStruct((N_NODES, w_out.shape[1]), jnp.float32),
    )(parts_a, parts_b, w1, b1.reshape(1, -1), w2, b2.reshape(1, -1),
      w3, b3.reshape(1, -1), w_out)


def kernel(x, rbf, i, num_nodes, W_rbf, W1, b1, W2, b2, W3, b3, W_out):
    e = x.shape[0]
    half = e // 2
    seg = jnp.minimum(i.astype(jnp.int32), jnp.int32(num_nodes) - 1)
    seg2d = seg.reshape(-1, _IDXROW)
    rbf_t = rbf.T
    zeros_nf = jnp.zeros((N_PAD, HIDDEN), jnp.float32)
    h0 = _edge_stage(rbf_t, x, W_rbf, 0, half)
    parts0 = _sc_scatter(h0, seg2d[: half // _IDXROW], zeros_nf)
    h1 = _edge_stage(rbf_t, x, W_rbf, half, half)
    parts1 = _sc_scatter(h1, seg2d[half // _IDXROW :], zeros_nf)
    return _mlp_stage(parts0, parts1, W1, b1, W2, b2, W3, b3, W_out)
